# Initial kernel scaffold; baseline (speedup 1.0000x reference)
import jax
import jax.numpy as jnp
from jax.experimental import pallas as pl


def kernel(n_x, e_x, up_x_i_idx, up_x_j_idx, up_attr, up_adj, up_index_b, up_index_i, up_index_j, n_x_idx_b, n_x_idx_n, boundary_adj, boundary_attr_idx_b, boundary_attr_idx_n, boundary_attr_mask, e_x_idx_b, e_x_idx_m, W_msg, b_msg, W_fb, b_fb, Pn, Pe):
    raise NotImplementedError("write your pallas kernel here")


# speedup vs baseline: 7.5018x; 7.5018x over previous
import functools

import jax
import jax.numpy as jnp
from jax.experimental import pallas as pl
from jax.experimental.pallas import tpu as pltpu

B, NMAX, MMAX, D, H = 128, 64, 96, 64, 64
N, M, E = B * NMAX, B * MMAX, 65536
NSLOT = B * NMAX * NMAX


def _edge_body(xi_ref, xj_ref, attr_ref, coeff_ref, wm_ref, wf_ref, bm_ref,
               bf_ref, out_ref):
    xi = xi_ref[...]
    xj = xj_ref[...]
    at = attr_ref[...]
    wm = wm_ref[...]
    wf = wf_ref[...]
    pre_m = (xj @ wm[:D] + at @ wm[D:] + bm_ref[...])
    pre_f = (xi @ wf[:D] + xj @ wf[D:2 * D] + at @ wf[2 * D:] + bf_ref[...])
    msg = jnp.maximum(pre_m, 0.0) * jax.nn.sigmoid(pre_f)
    out_ref[...] = msg * coeff_ref[...]


def _edge_contrib(x_i, x_j, attr, coeff, W_msg, W_fb, b_msg, b_fb):
    CH = 4096
    grid = (E // CH,)
    return pl.pallas_call(
        _edge_body,
        grid=grid,
        in_specs=[
            pl.BlockSpec((CH, D), lambda i: (i, 0)),
            pl.BlockSpec((CH, D), lambda i: (i, 0)),
            pl.BlockSpec((CH, D), lambda i: (i, 0)),
            pl.BlockSpec((CH, 1), lambda i: (i, 0)),
            pl.BlockSpec((2 * D, D), lambda i: (0, 0)),
            pl.BlockSpec((3 * D, D), lambda i: (0, 0)),
            pl.BlockSpec((1, D), lambda i: (0, 0)),
            pl.BlockSpec((1, D), lambda i: (0, 0)),
        ],
        out_specs=pl.BlockSpec((CH, D), lambda i: (i, 0)),
        out_shape=jax.ShapeDtypeStruct((E, D), jnp.float32),
    )(x_i, x_j, attr, coeff, W_msg, W_fb, b_msg.reshape(1, D),
      b_fb.reshape(1, D))


def _eagg_body(adj_ref, x_ref, out_ref):
    out_ref[0] = jnp.dot(adj_ref[0], x_ref[0],
                         preferred_element_type=jnp.float32)


def _boundary_agg(boundary_adj, dense_b):
    return pl.pallas_call(
        _eagg_body,
        grid=(B,),
        in_specs=[
            pl.BlockSpec((1, MMAX, NMAX), lambda b: (b, 0, 0)),
            pl.BlockSpec((1, NMAX, D), lambda b: (b, 0, 0)),
        ],
        out_specs=pl.BlockSpec((1, MMAX, D), lambda b: (b, 0, 0)),
        out_shape=jax.ShapeDtypeStruct((B, MMAX, D), jnp.float32),
    )(boundary_adj, dense_b)


def _mlp_in_kernel(x, params):
    for W, b, g, be in params:
        x = jnp.dot(x, W, preferred_element_type=jnp.float32) + b
        m = jnp.mean(x, axis=0, keepdims=True)
        v = jnp.mean((x - m) ** 2, axis=0, keepdims=True)
        x = (x - m) * jax.lax.rsqrt(v + 1e-5) * g + be
        x = jnp.maximum(x, 0.0)
    return x


def _tail_body(*refs):
    nin_ref, ein_ref = refs[0], refs[1]
    pn = [tuple(r[...] for r in refs[2 + 4 * i:6 + 4 * i]) for i in range(3)]
    pe = [tuple(r[...] for r in refs[14 + 4 * i:18 + 4 * i]) for i in range(3)]
    on_ref, oe_ref = refs[26], refs[27]
    on_ref[...] = _mlp_in_kernel(nin_ref[...], pn)
    oe_ref[...] = _mlp_in_kernel(ein_ref[...], pe)


def _tail(n_in, e_in, Pn, Pe):
    def flat(P):
        out = []
        for i in (1, 2, 3):
            out += [P['W%d' % i], P['b%d' % i].reshape(1, H),
                    P['g%d' % i].reshape(1, H), P['be%d' % i].reshape(1, H)]
        return out

    args = [n_in, e_in] + flat(Pn) + flat(Pe)
    return pl.pallas_call(
        _tail_body,
        out_shape=(jax.ShapeDtypeStruct((N, H), jnp.float32),
                   jax.ShapeDtypeStruct((M, H), jnp.float32)),
    )(*args)


def kernel(n_x, e_x, up_x_i_idx, up_x_j_idx, up_attr, up_adj, up_index_b,
           up_index_i, up_index_j, n_x_idx_b, n_x_idx_n, boundary_adj,
           boundary_attr_idx_b, boundary_attr_idx_n, boundary_attr_mask,
           e_x_idx_b, e_x_idx_m, W_msg, b_msg, W_fb, b_fb, Pn, Pe):
    eb = up_index_b.astype(jnp.int32)
    ei = up_index_i.astype(jnp.int32)
    ej = up_index_j.astype(jnp.int32)
    slot = (eb * NMAX + ei) * NMAX + ej
    row = slot // NMAX
    eid = jnp.arange(E, dtype=jnp.int32)

    last = jnp.full((NSLOT,), -1, jnp.int32).at[slot].max(eid)
    live = last[slot] == eid
    coeff = jnp.where(live, up_adj.reshape(-1)[slot], 0.0)

    x_i = n_x[up_x_i_idx]
    x_j = n_x[up_x_j_idx]
    contrib = _edge_contrib(x_i, x_j, up_attr, coeff.reshape(E, 1),
                            W_msg, W_fb, b_msg, b_fb)
    n_agg = jnp.zeros((N, D), jnp.float32).at[row].add(contrib)

    dense_b = n_x.reshape(B, NMAX, D)
    e_agg = _boundary_agg(boundary_adj, dense_b).reshape(M, D)

    return _tail(n_agg + n_x, e_agg + e_x, Pn, Pe)


# speedup vs baseline: 8.1212x; 1.0826x over previous
import functools

import jax
import jax.numpy as jnp
from jax import lax
from jax.experimental import pallas as pl
from jax.experimental.pallas import tpu as pltpu
from jax.experimental.pallas import tpu_sc as plsc

B, NMAX, MMAX, D, H = 128, 64, 96, 64, 64
N, M, E = B * NMAX, B * MMAX, 65536
NSLOT = B * NMAX * NMAX

NC, NS = 2, 16
NW = NC * NS
_MESH = functools.partial(plsc.VectorSubcoreMesh,
                          core_axis_name="c", subcore_axis_name="s",
                          num_cores=NC, num_subcores=NS)


def _sc_scatter_add(contrib, row2d, zeros):
    EPW = E // NW
    CH = 512
    KB = CH // 128

    @functools.partial(
        pl.kernel,
        out_type=jax.ShapeDtypeStruct((NC, N, D), jnp.float32),
        mesh=_MESH(),
        scratch_types=[
            pltpu.VMEM((CH, D), jnp.float32),
            pltpu.VMEM((KB, 128), jnp.int32),
            pltpu.VMEM_SHARED((N, D), jnp.float32),
            pltpu.VMEM((CH, D), jnp.float32),
        ],
        compiler_params=pltpu.CompilerParams(use_tc_tiling_on_sc=False),
    )
    def body(contrib_hbm, row_hbm, zeros_hbm, out_hbm, cbuf, rbuf, acc, obuf):
        cid = lax.axis_index("c")
        sid = lax.axis_index("s")
        wid = sid * NC + cid
        stripe = pl.ds(sid * (N // NS), N // NS)
        pltpu.sync_copy(zeros_hbm.at[stripe], cbuf)
        pltpu.sync_copy(cbuf, acc.at[stripe])
        plsc.subcore_barrier()
        for ci in range(EPW // CH):
            off = pl.multiple_of(wid * EPW + ci * CH, CH)
            pltpu.sync_copy(contrib_hbm.at[pl.ds(off, CH)], cbuf)
            pltpu.sync_copy(
                row_hbm.at[pl.ds(pl.multiple_of(off // 128, KB), KB)], rbuf)
            for k in range(KB):
                pltpu.sync_copy(cbuf.at[pl.ds(k * 128, 128)],
                                acc.at[rbuf.at[k]], add=True)
        plsc.subcore_barrier()
        pltpu.sync_copy(acc.at[stripe], obuf)
        pltpu.sync_copy(obuf, out_hbm.at[cid].at[stripe])

    return body(contrib, row2d, zeros)


def _edge_body(xi_ref, xj_ref, attr_ref, coeff_ref, wm_ref, wf_ref, bm_ref,
               bf_ref, out_ref):
    xi = xi_ref[...]
    xj = xj_ref[...]
    at = attr_ref[...]
    wm = wm_ref[...]
    wf = wf_ref[...]
    pre_m = (xj @ wm[:D] + at @ wm[D:] + bm_ref[...])
    pre_f = (xi @ wf[:D] + xj @ wf[D:2 * D] + at @ wf[2 * D:] + bf_ref[...])
    msg = jnp.maximum(pre_m, 0.0) * jax.nn.sigmoid(pre_f)
    out_ref[...] = msg * coeff_ref[...]


def _edge_contrib(x_i, x_j, attr, coeff, W_msg, W_fb, b_msg, b_fb):
    CH = 4096
    grid = (E // CH,)
    return pl.pallas_call(
        _edge_body,
        grid=grid,
        in_specs=[
            pl.BlockSpec((CH, D), lambda i: (i, 0)),
            pl.BlockSpec((CH, D), lambda i: (i, 0)),
            pl.BlockSpec((CH, D), lambda i: (i, 0)),
            pl.BlockSpec((CH, 1), lambda i: (i, 0)),
            pl.BlockSpec((2 * D, D), lambda i: (0, 0)),
            pl.BlockSpec((3 * D, D), lambda i: (0, 0)),
            pl.BlockSpec((1, D), lambda i: (0, 0)),
            pl.BlockSpec((1, D), lambda i: (0, 0)),
        ],
        out_specs=pl.BlockSpec((CH, D), lambda i: (i, 0)),
        out_shape=jax.ShapeDtypeStruct((E, D), jnp.float32),
    )(x_i, x_j, attr, coeff, W_msg, W_fb, b_msg.reshape(1, D),
      b_fb.reshape(1, D))


def _eagg_body(adj_ref, x_ref, out_ref):
    out_ref[0] = jnp.dot(adj_ref[0], x_ref[0],
                         preferred_element_type=jnp.float32)


def _boundary_agg(boundary_adj, dense_b):
    return pl.pallas_call(
        _eagg_body,
        grid=(B,),
        in_specs=[
            pl.BlockSpec((1, MMAX, NMAX), lambda b: (b, 0, 0)),
            pl.BlockSpec((1, NMAX, D), lambda b: (b, 0, 0)),
        ],
        out_specs=pl.BlockSpec((1, MMAX, D), lambda b: (b, 0, 0)),
        out_shape=jax.ShapeDtypeStruct((B, MMAX, D), jnp.float32),
    )(boundary_adj, dense_b)


def _mlp_in_kernel(x, params):
    for W, b, g, be in params:
        x = jnp.dot(x, W, preferred_element_type=jnp.float32) + b
        m = jnp.mean(x, axis=0, keepdims=True)
        v = jnp.mean((x - m) ** 2, axis=0, keepdims=True)
        x = (x - m) * jax.lax.rsqrt(v + 1e-5) * g + be
        x = jnp.maximum(x, 0.0)
    return x


def _tail_body(*refs):
    nin_ref, ein_ref = refs[0], refs[1]
    pn = [tuple(r[...] for r in refs[2 + 4 * i:6 + 4 * i]) for i in range(3)]
    pe = [tuple(r[...] for r in refs[14 + 4 * i:18 + 4 * i]) for i in range(3)]
    on_ref, oe_ref = refs[26], refs[27]
    on_ref[...] = _mlp_in_kernel(nin_ref[...], pn)
    oe_ref[...] = _mlp_in_kernel(ein_ref[...], pe)


def _tail(n_in, e_in, Pn, Pe):
    def flat(P):
        out = []
        for i in (1, 2, 3):
            out += [P['W%d' % i], P['b%d' % i].reshape(1, H),
                    P['g%d' % i].reshape(1, H), P['be%d' % i].reshape(1, H)]
        return out

    args = [n_in, e_in] + flat(Pn) + flat(Pe)
    return pl.pallas_call(
        _tail_body,
        out_shape=(jax.ShapeDtypeStruct((N, H), jnp.float32),
                   jax.ShapeDtypeStruct((M, H), jnp.float32)),
    )(*args)


def kernel(n_x, e_x, up_x_i_idx, up_x_j_idx, up_attr, up_adj, up_index_b,
           up_index_i, up_index_j, n_x_idx_b, n_x_idx_n, boundary_adj,
           boundary_attr_idx_b, boundary_attr_idx_n, boundary_attr_mask,
           e_x_idx_b, e_x_idx_m, W_msg, b_msg, W_fb, b_fb, Pn, Pe):
    eb = up_index_b.astype(jnp.int32)
    ei = up_index_i.astype(jnp.int32)
    ej = up_index_j.astype(jnp.int32)
    slot = (eb * NMAX + ei) * NMAX + ej
    row = slot // NMAX
    eid = jnp.arange(E, dtype=jnp.int32)

    last = jnp.full((NSLOT,), -1, jnp.int32).at[slot].max(eid)
    live = last[slot] == eid
    coeff = jnp.where(live, up_adj.reshape(-1)[slot], 0.0)

    x_i = n_x[up_x_i_idx]
    x_j = n_x[up_x_j_idx]
    contrib = _edge_contrib(x_i, x_j, up_attr, coeff.reshape(E, 1),
                            W_msg, W_fb, b_msg, b_fb)
    acc = _sc_scatter_add(contrib, row.reshape(E // 128, 128),
                          jnp.zeros((N, D), jnp.float32))
    n_agg = acc[0] + acc[1]

    dense_b = n_x.reshape(B, NMAX, D)
    e_agg = _boundary_agg(boundary_adj, dense_b).reshape(M, D)

    return _tail(n_agg + n_x, e_agg + e_x, Pn, Pe)
